# baseline (speedup 1.0000x reference)
import functools

import jax
import jax.numpy as jnp
from jax import lax
from jax.experimental import pallas as pl
from jax.experimental.pallas import tpu as pltpu
from jax.experimental.pallas import tpu_sc as plsc

_VSIZE = 100000
_NCLASS = 2
_ALPHA = 1.0
_SEQ = 200
_BATCH = 4096

_NW = 32
_BC = _BATCH // _NW


_VBLK = 25600
_NVBLK = 4


def _logtab_body(c_ref, xy_ref, o_ref):
    x = xy_ref[...]
    row1 = lax.broadcasted_iota(jnp.int32, (_NCLASS, 1), 0)
    ylog = jnp.log(jnp.where(row1 == 0, c_ref[0], c_ref[1]) + _VSIZE * _ALPHA)
    s = jnp.log(x + _ALPHA) - ylog
    b = lax.bitcast_convert_type(s, jnp.int32)
    rb = b + jnp.int32(0x7FFF) + ((b >> 16) & jnp.int32(1))
    bits = (rb >> 16) & jnp.int32(0xFFFF)
    packed = bits[0:1, :] | (bits[1:2, :] << 16)
    o_ref[...] = packed.reshape(_VBLK)


_XCH = 40
_NXCH = _SEQ // _XCH


def _sc_body(tab_hbm, x_hbm, out0_hbm, out1_hbm, tab_v, x_v, res_v, spm,
             sem_t, sem_a, sem_b):
    c = lax.axis_index("c")
    s = lax.axis_index("s")
    wid = s * 2 + c
    b0 = wid * _BC

    sems = (sem_a, sem_b)
    handles = {0: pltpu.async_copy(
        x_hbm.at[pl.ds(0, _XCH), pl.ds(b0, _BC)], x_v.at[0], sem_a)}

    @pl.when(s == 0)
    def _():
        pltpu.async_copy(tab_hbm, spm, sem_t).wait()

    plsc.subcore_barrier()
    pltpu.async_copy(spm, tab_v, sem_t).wait()

    zero = jnp.zeros((16,), jnp.float32)
    accs = (zero,) * (2 * (_BC // 16))

    for g in range(_NXCH):
        if g + 1 < _NXCH:
            handles[(g + 1) % 2] = pltpu.async_copy(
                x_hbm.at[pl.ds((g + 1) * _XCH, _XCH), pl.ds(b0, _BC)],
                x_v.at[(g + 1) % 2], sems[(g + 1) % 2])
        handles[g % 2].wait()
        par = g % 2

        def step(t, accs, par=par):
            nxt = []
            for j in range(_BC // 16):
                idx = x_v[par, t, pl.ds(j * 16, 16)]
                v = plsc.load_gather(tab_v, [idx])
                f0 = plsc.bitcast(v << 16, jnp.float32)
                f1 = plsc.bitcast(v & jnp.int32(-65536), jnp.float32)
                nxt.append(accs[2 * j] + f0)
                nxt.append(accs[2 * j + 1] + f1)
            return tuple(nxt)

        accs = lax.fori_loop(0, _XCH, step, accs)

    for j in range(_BC // 16):
        res_v[0, pl.ds(16 * j, 16)] = accs[2 * j]
        res_v[1, pl.ds(16 * j, 16)] = accs[2 * j + 1]

    cp0 = pltpu.async_copy(res_v.at[0], out0_hbm.at[pl.ds(b0, _BC)], sem_a)
    cp1 = pltpu.async_copy(res_v.at[1], out1_hbm.at[pl.ds(b0, _BC)], sem_b)
    cp0.wait()
    cp1.wait()


_sc_gather_sum = functools.partial(
    pl.kernel,
    out_type=(jax.ShapeDtypeStruct((_BATCH,), jnp.float32),
              jax.ShapeDtypeStruct((_BATCH,), jnp.float32)),
    mesh=plsc.VectorSubcoreMesh(core_axis_name="c", subcore_axis_name="s"),
    compiler_params=pltpu.CompilerParams(needs_layout_passes=False),
    scratch_types=[
        pltpu.VMEM((_VSIZE,), jnp.int32),
        pltpu.VMEM((2, _XCH, _BC), jnp.int32),
        pltpu.VMEM((_NCLASS, _BC), jnp.float32),
        pltpu.VMEM_SHARED((_VSIZE,), jnp.int32),
        pltpu.SemaphoreType.DMA,
        pltpu.SemaphoreType.DMA,
        pltpu.SemaphoreType.DMA,
    ],
)(_sc_body)


def kernel(input, xycounts, ycounts):
    tab_i32 = pl.pallas_call(
        _logtab_body,
        grid=(_NVBLK,),
        out_shape=jax.ShapeDtypeStruct((_VSIZE,), jnp.int32),
        in_specs=[
            pl.BlockSpec(memory_space=pltpu.SMEM),
            pl.BlockSpec((_NCLASS, _VBLK), lambda i: (0, i)),
        ],
        out_specs=pl.BlockSpec((_VBLK,), lambda i: (i,)),
    )(ycounts.astype(jnp.float32), jnp.swapaxes(xycounts, 0, 1))
    x = input.astype(jnp.int32)
    out0, out1 = _sc_gather_sum(tab_i32, x)
    return jnp.concatenate([out0[:, None], out1[:, None]], axis=1)


# speedup vs baseline: 1.0297x; 1.0297x over previous
import functools

import jax
import jax.numpy as jnp
from jax import lax
from jax.experimental import pallas as pl
from jax.experimental.pallas import tpu as pltpu
from jax.experimental.pallas import tpu_sc as plsc

_VSIZE = 100000
_NCLASS = 2
_ALPHA = 1.0
_SEQ = 200
_BATCH = 4096

_NW = 32
_BC = _BATCH // _NW


_VBLK = 51200
_NVBLK = 2


def _logtab_body(c_ref, xy_ref, o_ref):
    x = xy_ref[...]
    row1 = lax.broadcasted_iota(jnp.int32, (_NCLASS, 1), 0)
    ylog = jnp.log(jnp.where(row1 == 0, c_ref[0], c_ref[1]) + _VSIZE * _ALPHA)
    s = jnp.log(x + _ALPHA) - ylog
    b = lax.bitcast_convert_type(s, jnp.int32)
    rb = b + jnp.int32(0x7FFF) + ((b >> 16) & jnp.int32(1))
    bits = (rb >> 16) & jnp.int32(0xFFFF)
    packed = bits[0:1, :] | (bits[1:2, :] << 16)
    o_ref[...] = packed.reshape(_VBLK)


_XCH = 40
_NXCH = _SEQ // _XCH


def _sc_body(tab_hbm, x_hbm, out0_hbm, out1_hbm, tab_v, x_v, res_v, spm,
             sem_t, sem_a, sem_b):
    c = lax.axis_index("c")
    s = lax.axis_index("s")
    wid = s * 2 + c
    b0 = wid * _BC

    sems = (sem_a, sem_b)
    handles = {0: pltpu.async_copy(
        x_hbm.at[pl.ds(0, _XCH), pl.ds(b0, _BC)], x_v.at[0], sem_a)}

    @pl.when(s == 0)
    def _():
        pltpu.async_copy(tab_hbm, spm, sem_t).wait()

    plsc.subcore_barrier()
    pltpu.async_copy(spm, tab_v, sem_t).wait()

    zero = jnp.zeros((16,), jnp.float32)
    accs = (zero,) * (2 * (_BC // 16))

    for g in range(_NXCH):
        if g + 1 < _NXCH:
            handles[(g + 1) % 2] = pltpu.async_copy(
                x_hbm.at[pl.ds((g + 1) * _XCH, _XCH), pl.ds(b0, _BC)],
                x_v.at[(g + 1) % 2], sems[(g + 1) % 2])
        handles[g % 2].wait()
        par = g % 2

        def step(t, accs, par=par):
            nxt = []
            for j in range(_BC // 16):
                idx = x_v[par, t, pl.ds(j * 16, 16)]
                v = plsc.load_gather(tab_v, [idx])
                f0 = plsc.bitcast(v << 16, jnp.float32)
                f1 = plsc.bitcast(v & jnp.int32(-65536), jnp.float32)
                nxt.append(accs[2 * j] + f0)
                nxt.append(accs[2 * j + 1] + f1)
            return tuple(nxt)

        accs = lax.fori_loop(0, _XCH, step, accs)

    for j in range(_BC // 16):
        res_v[0, pl.ds(16 * j, 16)] = accs[2 * j]
        res_v[1, pl.ds(16 * j, 16)] = accs[2 * j + 1]

    cp0 = pltpu.async_copy(res_v.at[0], out0_hbm.at[pl.ds(b0, _BC)], sem_a)
    cp1 = pltpu.async_copy(res_v.at[1], out1_hbm.at[pl.ds(b0, _BC)], sem_b)
    cp0.wait()
    cp1.wait()


_sc_gather_sum = functools.partial(
    pl.kernel,
    out_type=(jax.ShapeDtypeStruct((_BATCH,), jnp.float32),
              jax.ShapeDtypeStruct((_BATCH,), jnp.float32)),
    mesh=plsc.VectorSubcoreMesh(core_axis_name="c", subcore_axis_name="s"),
    compiler_params=pltpu.CompilerParams(needs_layout_passes=False),
    scratch_types=[
        pltpu.VMEM((_VSIZE,), jnp.int32),
        pltpu.VMEM((2, _XCH, _BC), jnp.int32),
        pltpu.VMEM((_NCLASS, _BC), jnp.float32),
        pltpu.VMEM_SHARED((_VSIZE,), jnp.int32),
        pltpu.SemaphoreType.DMA,
        pltpu.SemaphoreType.DMA,
        pltpu.SemaphoreType.DMA,
    ],
)(_sc_body)


def kernel(input, xycounts, ycounts):
    tab_i32 = pl.pallas_call(
        _logtab_body,
        grid=(_NVBLK,),
        out_shape=jax.ShapeDtypeStruct((_VSIZE,), jnp.int32),
        in_specs=[
            pl.BlockSpec(memory_space=pltpu.SMEM),
            pl.BlockSpec((_NCLASS, _VBLK), lambda i: (0, i)),
        ],
        out_specs=pl.BlockSpec((_VBLK,), lambda i: (i,)),
    )(ycounts.astype(jnp.float32), jnp.swapaxes(xycounts, 0, 1))
    x = input.astype(jnp.int32)
    out0, out1 = _sc_gather_sum(tab_i32, x)
    return jnp.concatenate([out0[:, None], out1[:, None]], axis=1)
